# Initial kernel scaffold; baseline (speedup 1.0000x reference)
import jax
import jax.numpy as jnp
from jax.experimental import pallas as pl


def kernel(cls_scores, box_deltas, anchors):
    raise NotImplementedError("write your pallas kernel here")


# speedup vs baseline: 14.5691x; 14.5691x over previous
import jax
import jax.numpy as jnp
from jax.experimental import pallas as pl

N = 20000
R, C = 160, 128
P = R * C
IMG_H, IMG_W = 512.0, 512.0
BOX_FILTER_THRESHOLD = 0.05
NMS_THRESHOLD = 0.5
POST_NMS_TOP_K = 100
NMS_MARGIN = 0.0
NEG_INF = -1e9


def _nms_kernel(cls_ref, del_ref, anc_ref, box_out, sc_out, lb_out):
    num_classes = cls_ref.shape[0]

    def cls_body(c, carry):
        best, lab = carry
        v = cls_ref[c]
        better = v > best
        return jnp.where(better, v, best), jnp.where(better, c, lab)

    best0 = cls_ref[0]
    lab0 = jnp.zeros((R, C), jnp.int32)
    best, labels = jax.lax.fori_loop(1, num_classes, cls_body, (best0, lab0))

    ax, ay, aw, ah = anc_ref[0], anc_ref[1], anc_ref[2], anc_ref[3]
    dx, dy, dw, dh = del_ref[0], del_ref[1], del_ref[2], del_ref[3]
    cx = ax + dx * aw
    cy = ay + dy * ah
    w = aw * jnp.exp(dw)
    h = ah * jnp.exp(dh)
    x1 = jnp.clip(cx - 0.5 * w, 0.0, IMG_W)
    y1 = jnp.clip(cy - 0.5 * h, 0.0, IMG_H)
    x2 = jnp.clip(cx + 0.5 * w, 0.0, IMG_W)
    y2 = jnp.clip(cy + 0.5 * h, 0.0, IMG_H)
    area = jnp.maximum(x2 - x1, 0.0) * jnp.maximum(y2 - y1, 0.0)

    sw0 = jnp.where(best >= BOX_FILTER_THRESHOLD, best, NEG_INF)

    lin = (jax.lax.broadcasted_iota(jnp.int32, (R, C), 0) * C
           + jax.lax.broadcasted_iota(jnp.int32, (R, C), 1))
    slin = (jax.lax.broadcasted_iota(jnp.int32, (8, 128), 0) * 128
            + jax.lax.broadcasted_iota(jnp.int32, (8, 128), 1))

    sc_out[...] = jnp.zeros((8, 128), jnp.float32)
    lb_out[...] = jnp.full((8, 128), -1, jnp.int32)
    for i in range(4):
        box_out[i] = jnp.zeros((8, 128), jnp.float32)

    def body(t, sw):
        s = jnp.max(sw)
        idx = jnp.min(jnp.where(sw == s, lin, jnp.int32(P)))
        hot = lin == idx
        bx1 = jnp.sum(jnp.where(hot, x1, 0.0))
        by1 = jnp.sum(jnp.where(hot, y1, 0.0))
        bx2 = jnp.sum(jnp.where(hot, x2, 0.0))
        by2 = jnp.sum(jnp.where(hot, y2, 0.0))
        area_a = jnp.sum(jnp.where(hot, area, 0.0))
        blab = jnp.sum(jnp.where(hot, labels, 0))
        valid = s > (NEG_INF / 2.0)

        ix1 = jnp.maximum(bx1, x1)
        iy1 = jnp.maximum(by1, y1)
        ix2 = jnp.minimum(bx2, x2)
        iy2 = jnp.minimum(by2, y2)
        inter = jnp.maximum(ix2 - ix1, 0.0) * jnp.maximum(iy2 - iy1, 0.0)
        iou = inter / (area_a + area - inter + 1e-9)
        supp = (iou > NMS_THRESHOLD) & ((s - sw) >= NMS_MARGIN) & valid
        sw = jnp.where(supp | hot, NEG_INF, sw)

        hot_t = slin == t
        sc_out[...] = jnp.where(hot_t, jnp.where(valid, s, 0.0), sc_out[...])
        lb_out[...] = jnp.where(hot_t, jnp.where(valid, blab, -1), lb_out[...])
        bvals = (bx1, by1, bx2, by2)
        for i in range(4):
            box_out[i] = jnp.where(hot_t, jnp.where(valid, bvals[i], 0.0),
                                   box_out[i])
        return sw

    jax.lax.fori_loop(0, POST_NMS_TOP_K, body, sw0)


def kernel(cls_scores, box_deltas, anchors):
    n, num_classes = cls_scores.shape
    pad = P - n
    cls_t = jnp.pad(cls_scores, ((0, pad), (0, 0)),
                    constant_values=-1.0).T.reshape(num_classes, R, C)
    del_t = jnp.pad(box_deltas, ((0, pad), (0, 0))).T.reshape(4, R, C)
    anc_t = jnp.pad(anchors, ((0, pad), (0, 0))).T.reshape(4, R, C)

    bx, sc, lb = pl.pallas_call(
        _nms_kernel,
        out_shape=(
            jax.ShapeDtypeStruct((4, 8, 128), jnp.float32),
            jax.ShapeDtypeStruct((8, 128), jnp.float32),
            jax.ShapeDtypeStruct((8, 128), jnp.int32),
        ),
    )(cls_t, del_t, anc_t)

    boxes = bx.reshape(4, 8 * 128)[:, :POST_NMS_TOP_K].T
    scores = sc.reshape(8 * 128)[:POST_NMS_TOP_K]
    labels = lb.reshape(8 * 128)[:POST_NMS_TOP_K]
    return boxes, scores, labels


# speedup vs baseline: 15.1937x; 1.0429x over previous
import jax
import jax.numpy as jnp
from jax.experimental import pallas as pl
from jax.experimental.pallas import tpu as pltpu

N = 20000
R, C = 160, 128
P = R * C
IMG_H, IMG_W = 512.0, 512.0
BOX_FILTER_THRESHOLD = 0.05
NMS_THRESHOLD = 0.5
POST_NMS_TOP_K = 100
NEG_INF = -1e9


def _nms_kernel(cls_ref, del_ref, anc_ref, box_out, sc_out, lb_out,
                x1_ref, y1_ref, x2_ref, y2_ref, area_ref, lab_ref):
    num_classes = cls_ref.shape[0]

    def cls_body(c, carry):
        best, lab = carry
        v = cls_ref[c]
        better = v > best
        return jnp.where(better, v, best), jnp.where(better, c, lab)

    best0 = cls_ref[0]
    lab0 = jnp.zeros((R, C), jnp.int32)
    best, labels = jax.lax.fori_loop(1, num_classes, cls_body, (best0, lab0))
    lab_ref[...] = labels

    ax, ay, aw, ah = anc_ref[0], anc_ref[1], anc_ref[2], anc_ref[3]
    dx, dy, dw, dh = del_ref[0], del_ref[1], del_ref[2], del_ref[3]
    cx = ax + dx * aw
    cy = ay + dy * ah
    w = aw * jnp.exp(dw)
    h = ah * jnp.exp(dh)
    x1 = jnp.clip(cx - 0.5 * w, 0.0, IMG_W)
    y1 = jnp.clip(cy - 0.5 * h, 0.0, IMG_H)
    x2 = jnp.clip(cx + 0.5 * w, 0.0, IMG_W)
    y2 = jnp.clip(cy + 0.5 * h, 0.0, IMG_H)
    x1_ref[...] = x1
    y1_ref[...] = y1
    x2_ref[...] = x2
    y2_ref[...] = y2
    area_ref[...] = jnp.maximum(x2 - x1, 0.0) * jnp.maximum(y2 - y1, 0.0)

    sw0 = jnp.where(best >= BOX_FILTER_THRESHOLD, best, NEG_INF)

    lin = (jax.lax.broadcasted_iota(jnp.int32, (R, C), 0) * C
           + jax.lax.broadcasted_iota(jnp.int32, (R, C), 1))
    lane_iota = jax.lax.broadcasted_iota(jnp.int32, (1, C), 1)
    slin = (jax.lax.broadcasted_iota(jnp.int32, (8, 128), 0) * 128
            + jax.lax.broadcasted_iota(jnp.int32, (8, 128), 1))

    sc_out[...] = jnp.zeros((8, 128), jnp.float32)
    lb_out[...] = jnp.full((8, 128), -1, jnp.int32)
    for i in range(4):
        box_out[i] = jnp.zeros((8, 128), jnp.float32)

    def body(t, sw):
        s = jnp.max(sw)
        idx = jnp.min(jnp.where(sw == s, lin, jnp.int32(P)))
        row = idx // C
        lane_hot = lane_iota == (idx - row * C)

        def ext(ref, zero):
            return jnp.sum(jnp.where(lane_hot, ref[pl.ds(row, 1), :], zero))

        bx1 = ext(x1_ref, 0.0)
        by1 = ext(y1_ref, 0.0)
        bx2 = ext(x2_ref, 0.0)
        by2 = ext(y2_ref, 0.0)
        blab = ext(lab_ref, 0)
        area_a = jnp.maximum(bx2 - bx1, 0.0) * jnp.maximum(by2 - by1, 0.0)
        valid = s > (NEG_INF / 2.0)

        inter = (jnp.maximum(jnp.minimum(bx2, x2_ref[...])
                             - jnp.maximum(bx1, x1_ref[...]), 0.0)
                 * jnp.maximum(jnp.minimum(by2, y2_ref[...])
                               - jnp.maximum(by1, y1_ref[...]), 0.0))
        iou = inter / (area_a + area_ref[...] - inter + 1e-9)
        supp = (iou > NMS_THRESHOLD) & valid
        sw = jnp.where(supp | (lin == idx), NEG_INF, sw)

        hot_t = slin == t
        sc_out[...] = jnp.where(hot_t, jnp.where(valid, s, 0.0), sc_out[...])
        lb_out[...] = jnp.where(hot_t, jnp.where(valid, blab, -1), lb_out[...])
        bvals = (bx1, by1, bx2, by2)
        for i in range(4):
            box_out[i] = jnp.where(hot_t, jnp.where(valid, bvals[i], 0.0),
                                   box_out[i])
        return sw

    jax.lax.fori_loop(0, POST_NMS_TOP_K, body, sw0)


def kernel(cls_scores, box_deltas, anchors):
    n, num_classes = cls_scores.shape
    pad = P - n
    cls_t = jnp.pad(cls_scores, ((0, pad), (0, 0)),
                    constant_values=-1.0).T.reshape(num_classes, R, C)
    del_t = jnp.pad(box_deltas, ((0, pad), (0, 0))).T.reshape(4, R, C)
    anc_t = jnp.pad(anchors, ((0, pad), (0, 0))).T.reshape(4, R, C)

    bx, sc, lb = pl.pallas_call(
        _nms_kernel,
        out_shape=(
            jax.ShapeDtypeStruct((4, 8, 128), jnp.float32),
            jax.ShapeDtypeStruct((8, 128), jnp.float32),
            jax.ShapeDtypeStruct((8, 128), jnp.int32),
        ),
        scratch_shapes=[
            pltpu.VMEM((R, C), jnp.float32),
            pltpu.VMEM((R, C), jnp.float32),
            pltpu.VMEM((R, C), jnp.float32),
            pltpu.VMEM((R, C), jnp.float32),
            pltpu.VMEM((R, C), jnp.float32),
            pltpu.VMEM((R, C), jnp.int32),
        ],
    )(cls_t, del_t, anc_t)

    boxes = bx.reshape(4, 8 * 128)[:, :POST_NMS_TOP_K].T
    scores = sc.reshape(8 * 128)[:POST_NMS_TOP_K]
    labels = lb.reshape(8 * 128)[:POST_NMS_TOP_K]
    return boxes, scores, labels
